# baseline (speedup 1.0000x reference)
import functools

import jax
import jax.numpy as jnp
import numpy as np
from jax import lax
from jax.experimental import pallas as pl
from jax.experimental.pallas import tpu as pltpu
from jax.experimental.pallas import tpu_sc as plsc

_N = 16384 * 200
_B = 16384

_info = plsc.get_sparse_core_info()
_NC = _info.num_cores
_NS = _info.num_subcores
_L = _info.num_lanes
_NW = _NC * _NS

_EPW = _N // _NW
_CHUNKE = 10240
_NCHUNK = _EPW // _CHUNKE
_UNROLL = 16
_STEPS = _CHUNKE // (_L * _UNROLL)
_NBUF = 3


def _sc_gather_dot(xf, idx, label):
    mesh = plsc.VectorSubcoreMesh(core_axis_name="c", subcore_axis_name="s")

    @functools.partial(
        pl.kernel,
        mesh=mesh,
        out_type=jax.ShapeDtypeStruct((_NW, _L), jnp.float32),
        scratch_types=(
            [pltpu.VMEM_SHARED((_B,), jnp.float32)]
            + [pltpu.VMEM((_CHUNKE,), jnp.float32)] * _NBUF
            + [pltpu.VMEM((_CHUNKE,), jnp.int32)] * _NBUF
            + [pltpu.VMEM((_CHUNKE,), jnp.float32)] * _NBUF
            + [pltpu.VMEM((_L,), jnp.float32)]
            + [pltpu.SemaphoreType.DMA] * _NBUF
            + [pltpu.SemaphoreType.DMA] * _NBUF
        ),
        compiler_params=pltpu.CompilerParams(needs_layout_passes=False),
    )
    def body(x_hbm, idx_hbm, label_hbm, out_hbm, label_v,
             x0, x1, x2, i0, i1, i2, g0, g1, g2, acc_v,
             si0, si1, si2, sg0, sg1, sg2):
        wid = lax.axis_index("s") * _NC + lax.axis_index("c")
        ebase = wid * _EPW
        xbufs = (x0, x1, x2)
        ibufs = (i0, i1, i2)
        gbufs = (g0, g1, g2)
        sin = (si0, si1, si2)
        sg = (sg0, sg1, sg2)

        @pl.when(lax.axis_index("s") == 0)
        def _load_table():
            pltpu.sync_copy(label_hbm, label_v)

        plsc.subcore_barrier()

        def start_in(c):
            src = pl.ds(ebase + c * _CHUNKE, _CHUNKE)
            b = c % _NBUF
            return (pltpu.async_copy(x_hbm.at[src], xbufs[b], sin[b]),
                    pltpu.async_copy(idx_hbm.at[src], ibufs[b], sin[b]))

        def start_gather(c):
            b = c % _NBUF
            return pltpu.async_copy(label_v.at[ibufs[b]], gbufs[b], sg[b])

        pend_in = {0: start_in(0), 1: start_in(1)}
        pend_g = {}
        acc = jnp.zeros((_L,), jnp.float32)
        for c in range(_NCHUNK + 1):
            if c < _NCHUNK:
                for h in pend_in.pop(c):
                    h.wait()
                pend_g[c] = start_gather(c)
            if c >= 1:
                pend_g.pop(c - 1).wait()
                x_v = xbufs[(c - 1) % _NBUF]
                g_v = gbufs[(c - 1) % _NBUF]

                def step(r, acc):
                    for u in range(_UNROLL):
                        o = r * _L * _UNROLL + u * _L
                        acc = acc + x_v[pl.ds(o, _L)] * g_v[pl.ds(o, _L)]
                    return acc

                acc = lax.fori_loop(0, _STEPS, step, acc)
            if c + 2 < _NCHUNK:
                pend_in[c + 2] = start_in(c + 2)

        acc_v[...] = acc
        pltpu.sync_copy(acc_v, out_hbm.at[wid])

    return body(xf, idx, label)


_TC_GRID = 8
_TC_BLK = _N // _TC_GRID


def _tc_dense_body(x_ref, o_ref):
    v = x_ref[...].reshape(_TC_BLK // 128, 128)
    val = jnp.maximum(v, 0.0) + jnp.log1p(jnp.exp(-jnp.abs(v)))
    s = jnp.sum(val, axis=0, keepdims=True)

    @pl.when(pl.program_id(0) == 0)
    def _init():
        o_ref[...] = s

    @pl.when(pl.program_id(0) != 0)
    def _acc():
        o_ref[...] += s


def _tc_dense_sum(xf):
    return pl.pallas_call(
        _tc_dense_body,
        grid=(_TC_GRID,),
        in_specs=[pl.BlockSpec((_TC_BLK,), lambda i: (i,))],
        out_specs=pl.BlockSpec((1, 128), lambda i: (0, 0)),
        out_shape=jax.ShapeDtypeStruct((1, 128), jnp.float32),
    )(xf)


def kernel(X, batch_idx, label):
    xf = X.reshape(_N)
    sc_parts = _sc_gather_dot(xf, batch_idx.astype(jnp.int32), label)
    tc_parts = _tc_dense_sum(xf)
    total = jnp.sum(tc_parts) - jnp.sum(sc_parts)
    return total * np.float32(1.0 / _N)


# speedup vs baseline: 2.5687x; 2.5687x over previous
import functools

import jax
import jax.numpy as jnp
import numpy as np
from jax import lax
from jax.experimental import pallas as pl
from jax.experimental.pallas import tpu as pltpu
from jax.experimental.pallas import tpu_sc as plsc

_N = 16384 * 200
_B = 16384

_info = plsc.get_sparse_core_info()
_NC = _info.num_cores
_NS = _info.num_subcores
_L = _info.num_lanes
_NW = _NC * _NS

_EPW = _N // _NW
_CHUNKE = 20480
_NCHUNK = _EPW // _CHUNKE
_UNROLL = 16
_STEPS = _CHUNKE // (_L * _UNROLL)


def _sc_gather_dot(xf, idx, label):
    mesh = plsc.VectorSubcoreMesh(core_axis_name="c", subcore_axis_name="s")

    @functools.partial(
        pl.kernel,
        mesh=mesh,
        out_type=jax.ShapeDtypeStruct((_NW, _L), jnp.float32),
        scratch_types=[
            pltpu.VMEM((_B,), jnp.float32),
            pltpu.VMEM((_CHUNKE,), jnp.float32),
            pltpu.VMEM((_CHUNKE,), jnp.float32),
            pltpu.VMEM((_CHUNKE,), jnp.int32),
            pltpu.VMEM((_CHUNKE,), jnp.int32),
            pltpu.VMEM((_L,), jnp.float32),
            pltpu.SemaphoreType.DMA,
            pltpu.SemaphoreType.DMA,
        ],
        compiler_params=pltpu.CompilerParams(needs_layout_passes=False),
    )
    def body(x_hbm, idx_hbm, label_hbm, out_hbm,
             label_v, x0_v, x1_v, i0_v, i1_v, acc_v, sem0, sem1):
        wid = lax.axis_index("s") * _NC + lax.axis_index("c")
        ebase = wid * _EPW
        xbufs = (x0_v, x1_v)
        ibufs = (i0_v, i1_v)
        sems = (sem0, sem1)
        pltpu.sync_copy(label_hbm, label_v)

        def start(c):
            src = pl.ds(ebase + c * _CHUNKE, _CHUNKE)
            b = c % 2
            return (pltpu.async_copy(x_hbm.at[src], xbufs[b], sems[b]),
                    pltpu.async_copy(idx_hbm.at[src], ibufs[b], sems[b]))

        pending = start(0)
        accs = (jnp.zeros((_L,), jnp.float32),) * 4
        for c in range(_NCHUNK):
            for h in pending:
                h.wait()
            if c + 1 < _NCHUNK:
                pending = start(c + 1)
            x_v = xbufs[c % 2]
            idx_v = ibufs[c % 2]

            def step(r, accs):
                accs = list(accs)
                for u in range(_UNROLL):
                    o = r * _L * _UNROLL + u * _L
                    xv = x_v[pl.ds(o, _L)]
                    iv = idx_v[pl.ds(o, _L)]
                    g = plsc.load_gather(label_v, [iv])
                    accs[u % 4] = accs[u % 4] + xv * g
                return tuple(accs)

            accs = lax.fori_loop(0, _STEPS, step, accs)

        acc_v[...] = (accs[0] + accs[1]) + (accs[2] + accs[3])
        pltpu.sync_copy(acc_v, out_hbm.at[wid])

    return body(xf, idx, label)


_TC_GRID = 8
_TC_BLK = _N // _TC_GRID


def _tc_dense_body(x_ref, o_ref):
    v = x_ref[...].reshape(_TC_BLK // 128, 128)
    val = jnp.maximum(v, 0.0) + jnp.log1p(jnp.exp(-jnp.abs(v)))
    s = jnp.sum(val, axis=0, keepdims=True)

    @pl.when(pl.program_id(0) == 0)
    def _init():
        o_ref[...] = s

    @pl.when(pl.program_id(0) != 0)
    def _acc():
        o_ref[...] += s


def _tc_dense_sum(xf):
    return pl.pallas_call(
        _tc_dense_body,
        grid=(_TC_GRID,),
        in_specs=[pl.BlockSpec((_TC_BLK,), lambda i: (i,))],
        out_specs=pl.BlockSpec((1, 128), lambda i: (0, 0)),
        out_shape=jax.ShapeDtypeStruct((1, 128), jnp.float32),
    )(xf)


def kernel(X, batch_idx, label):
    xf = X.reshape(_N)
    sc_parts = _sc_gather_dot(xf, batch_idx.astype(jnp.int32), label)
    tc_parts = _tc_dense_sum(xf)
    total = jnp.sum(tc_parts) - jnp.sum(sc_parts)
    return total * np.float32(1.0 / _N)
